# Initial kernel scaffold; baseline (speedup 1.0000x reference)
import jax
import jax.numpy as jnp
from jax.experimental import pallas as pl


def kernel(x, skip, w_up, b_up, w1, b1, g1, beta1, m1, v1, w2, b2, g2, beta2, m2, v2):
    raise NotImplementedError("write your pallas kernel here")


# speedup vs baseline: 1.3801x; 1.3801x over previous
import jax
import jax.numpy as jnp
from jax import lax
from jax.experimental import pallas as pl
from jax.experimental.pallas import tpu as pltpu


def kernel(x, skip, w_up, b_up, w1, b1, g1, beta1, m1, v1,
           w2, b2, g2, beta2, m2, v2):
    f32, bf16 = jnp.float32, jnp.bfloat16
    N, Cin, H, W = x.shape
    C = b_up.shape[0]
    H2, W2 = 2 * H, 2 * W
    WP = W2 + 4
    RP = H2 + 6
    M1 = (H2 + 2) * WP
    M2 = H2 * WP
    YR = (H2 + 4) * WP
    eps = 1e-5

    xt = jnp.transpose(x, (0, 2, 3, 1))
    zx = jnp.zeros_like(xt)
    xd = jnp.stack([jnp.concatenate([xt, zx], -1),
                    jnp.concatenate([zx, xt], -1)], axis=3)
    xd = xd.reshape(N, H * 2 * W, 2 * Cin).astype(bf16)

    skip_f = jnp.transpose(skip, (0, 2, 3, 1))
    skip_f = jnp.pad(skip_f, ((0, 0), (3, 3), (2, 2), (0, 0)))
    skip_f = skip_f.reshape(N, RP * WP, C).astype(bf16)

    wu = jnp.transpose(w_up, (3, 0, 2, 1)).reshape(2 * Cin, 2 * C)
    wu = wu.astype(bf16)
    bvec = jnp.tile(b_up.astype(f32), 2)[None, :]

    sc1 = g1 / jnp.sqrt(v1 + eps)
    sh1 = ((b1 - m1) * sc1 + beta1).astype(f32)[None, :]
    sc2 = g2 / jnp.sqrt(v2 + eps)
    sh2 = ((b2 - m2) * sc2 + beta2).astype(f32)[None, :]

    w1t = jnp.transpose(w1, (2, 3, 1, 0)) * sc1[None, None, None, :]
    w1t = w1t.reshape(9, 2 * C, C).astype(bf16)

    w2t = jnp.transpose(w2, (2, 3, 1, 0)) * sc2[None, None, None, :]
    w2a = jnp.concatenate([w2t[0], w2t[1]], axis=1).astype(bf16)
    w2b = w2t[2].astype(bf16)

    def body(xd_ref, skip_ref, wu_ref, bv_ref, w1_ref, w2a_ref, w2b_ref,
             sh1_ref, sh2_ref, o_ref, up_buf, y1_buf):
        up_buf[...] = jnp.zeros((RP * WP, C), bf16)
        y1_buf[0:WP, :] = jnp.zeros((WP, 2 * C), bf16)
        y1_buf[WP + M1:YR, :] = jnp.zeros((WP, 2 * C), bf16)

        bv = bv_ref[...]
        CH = min(8, H)
        rows = CH * 2 * W
        for hc in range(H // CH):
            zc = jnp.dot(xd_ref[0, hc * rows:(hc + 1) * rows, :], wu_ref[...],
                         preferred_element_type=f32) + bv
            zb = zc.astype(bf16)
            for hh in range(CH):
                h = hc * CH + hh
                for dy in range(2):
                    r0 = (3 + 2 * h + dy) * WP + 2
                    up_buf[r0:r0 + 2 * W, :] = \
                        zb[hh * 2 * W:(hh + 1) * 2 * W, dy * C:(dy + 1) * C]

        acc = None
        for kh in range(3):
            for kw in range(3):
                off = (kh + 1) * WP + kw - 1
                lhs = jnp.concatenate([skip_ref[0, off:off + M1, :],
                                       up_buf[off:off + M1, :]], axis=1)
                d = jnp.dot(lhs, w1_ref[kh * 3 + kw],
                            preferred_element_type=f32)
                acc = d if acc is None else acc + d
        y1 = jnp.maximum(acc + sh1_ref[...], 0.0)
        fi = lax.broadcasted_iota(jnp.int32, (M1, 1), 0)
        rr = fi // WP
        cc = fi - rr * WP
        ok = jnp.logical_and(
            jnp.logical_and(rr >= 1, rr < H2 + 1),
            jnp.logical_and(cc >= 2, cc < W2 + 2))
        y1b = jnp.where(ok, y1, 0.0).astype(bf16)
        y1_buf[WP:WP + M1, 0:C] = y1b
        y1_buf[0:M1, C:2 * C] = y1b

        acc2 = None
        for kw in range(3):
            off = WP + kw - 1
            d = jnp.dot(y1_buf[off:off + M2, :], w2a_ref[kw],
                        preferred_element_type=f32)
            acc2 = d if acc2 is None else acc2 + d
        for kw in range(3):
            off = 3 * WP + kw - 1
            acc2 = acc2 + jnp.dot(y1_buf[off:off + M2, 0:C], w2b_ref[kw],
                                  preferred_element_type=f32)
        o_ref[0] = jnp.maximum(acc2 + sh2_ref[...], 0.0)

    out_f = pl.pallas_call(
        body,
        out_shape=jax.ShapeDtypeStruct((N, M2, C), f32),
        grid_spec=pltpu.PrefetchScalarGridSpec(
            num_scalar_prefetch=0,
            grid=(N,),
            in_specs=[
                pl.BlockSpec((1, H * 2 * W, 2 * Cin), lambda n: (n, 0, 0)),
                pl.BlockSpec((1, RP * WP, C), lambda n: (n, 0, 0)),
                pl.BlockSpec((2 * Cin, 2 * C), lambda n: (0, 0)),
                pl.BlockSpec((1, 2 * C), lambda n: (0, 0)),
                pl.BlockSpec((9, 2 * C, C), lambda n: (0, 0, 0)),
                pl.BlockSpec((3, 2 * C, C), lambda n: (0, 0, 0)),
                pl.BlockSpec((3, C, C), lambda n: (0, 0, 0)),
                pl.BlockSpec((1, C), lambda n: (0, 0)),
                pl.BlockSpec((1, C), lambda n: (0, 0)),
            ],
            out_specs=pl.BlockSpec((1, M2, C), lambda n: (n, 0, 0)),
            scratch_shapes=[
                pltpu.VMEM((RP * WP, C), bf16),
                pltpu.VMEM((YR, 2 * C), bf16),
            ]),
        compiler_params=pltpu.CompilerParams(
            dimension_semantics=("parallel",),
            vmem_limit_bytes=64 * 1024 * 1024),
    )(xd, skip_f, wu, bvec, w1t, w2a, w2b, sh1, sh2)

    out = out_f.reshape(N, H2, WP, C)[:, :, 2:2 + W2, :]
    return jnp.transpose(out, (0, 3, 1, 2))


# speedup vs baseline: 2.2736x; 1.6475x over previous
import jax
import jax.numpy as jnp
from jax import lax
from jax.experimental import pallas as pl
from jax.experimental.pallas import tpu as pltpu


def kernel(x, skip, w_up, b_up, w1, b1, g1, beta1, m1, v1,
           w2, b2, g2, beta2, m2, v2):
    f32, bf16 = jnp.float32, jnp.bfloat16
    N, Cin, H, W = x.shape
    C = b_up.shape[0]
    H2, W2 = 2 * H, 2 * W
    WP = W2 + 4
    RP = H2 + 6
    M1 = (H2 + 2) * WP
    M2 = H2 * WP
    YR = (H2 + 4) * WP
    P = H2 * W2
    eps = 1e-5

    x_r = x.reshape(N, Cin, H * W)
    skip_r = skip.reshape(N, C, P)

    wu = jnp.transpose(w_up, (0, 2, 3, 1)).reshape(Cin, 4 * C).astype(bf16)
    bvec = jnp.tile(b_up.astype(f32), 4)[None, :]

    sc1 = g1 / jnp.sqrt(v1 + eps)
    sh1 = ((b1 - m1) * sc1 + beta1).astype(f32)[None, :]
    sc2 = g2 / jnp.sqrt(v2 + eps)
    sh2 = ((b2 - m2) * sc2 + beta2).astype(f32)[None, :]

    w1t = jnp.transpose(w1, (2, 3, 1, 0)) * sc1[None, None, None, :]
    w1t = w1t.reshape(9, 2 * C, C).astype(bf16)

    w2t = jnp.transpose(w2, (2, 3, 1, 0)) * sc2[None, None, None, :]
    w2a = jnp.concatenate([w2t[0], w2t[1]], axis=1).astype(bf16)
    w2b = w2t[2].astype(bf16)

    def body(x_ref, skip_ref, wu_ref, bv_ref, w1_ref, w2a_ref, w2b_ref,
             sh1_ref, sh2_ref, o_ref, up_buf, skip_buf, y1_buf):
        up_buf[...] = jnp.zeros((RP * WP, C), bf16)
        skip_buf[...] = jnp.zeros((RP * WP, C), bf16)
        y1_buf[0:WP, :] = jnp.zeros((WP, 2 * C), bf16)
        y1_buf[WP + M1:YR, :] = jnp.zeros((WP, 2 * C), bf16)

        skT = jnp.transpose(skip_ref[0]).astype(bf16)
        for r in range(H2):
            skip_buf[(3 + r) * WP + 2:(3 + r) * WP + 2 + W2, :] = \
                skT[r * W2:(r + 1) * W2, :]

        xb = x_ref[0].astype(bf16)
        zc = lax.dot_general(xb, wu_ref[...],
                             (((0,), (0,)), ((), ())),
                             preferred_element_type=f32) + bv_ref[...]
        zb = zc.astype(bf16)
        for h in range(H):
            for dy in range(2):
                r0 = (3 + 2 * h + dy) * WP + 2
                up_buf[r0:r0 + 2 * W, :] = \
                    zb[h * W:(h + 1) * W,
                       dy * 2 * C:(dy + 1) * 2 * C].reshape(2 * W, C)

        acc = None
        for kh in range(3):
            for kw in range(3):
                off = (kh + 1) * WP + kw - 1
                lhs = jnp.concatenate([skip_buf[off:off + M1, :],
                                       up_buf[off:off + M1, :]], axis=1)
                d = jnp.dot(lhs, w1_ref[kh * 3 + kw],
                            preferred_element_type=f32)
                acc = d if acc is None else acc + d
        y1 = jnp.maximum(acc + sh1_ref[...], 0.0)
        fi = lax.broadcasted_iota(jnp.int32, (M1, 1), 0)
        rr = fi // WP
        cc = fi - rr * WP
        ok = jnp.logical_and(
            jnp.logical_and(rr >= 1, rr < H2 + 1),
            jnp.logical_and(cc >= 2, cc < W2 + 2))
        y1b = jnp.where(ok, y1, 0.0).astype(bf16)
        y1_buf[WP:WP + M1, 0:C] = y1b
        y1_buf[0:M1, C:2 * C] = y1b

        acc2 = None
        for kw in range(3):
            off = WP + kw - 1
            d = jnp.dot(y1_buf[off:off + M2, :], w2a_ref[kw],
                        preferred_element_type=f32)
            acc2 = d if acc2 is None else acc2 + d
        for kw in range(3):
            off = 3 * WP + kw - 1
            acc2 = acc2 + jnp.dot(y1_buf[off:off + M2, 0:C], w2b_ref[kw],
                                  preferred_element_type=f32)
        y2 = jnp.maximum(acc2 + sh2_ref[...], 0.0)

        dense = jnp.concatenate(
            [y2[r * WP + 2:r * WP + 2 + W2, :] for r in range(H2)], axis=0)
        o_ref[0] = jnp.transpose(dense)

    out_r = pl.pallas_call(
        body,
        out_shape=jax.ShapeDtypeStruct((N, C, P), f32),
        grid_spec=pltpu.PrefetchScalarGridSpec(
            num_scalar_prefetch=0,
            grid=(N,),
            in_specs=[
                pl.BlockSpec((1, Cin, H * W), lambda n: (n, 0, 0)),
                pl.BlockSpec((1, C, P), lambda n: (n, 0, 0)),
                pl.BlockSpec((Cin, 4 * C), lambda n: (0, 0)),
                pl.BlockSpec((1, 4 * C), lambda n: (0, 0)),
                pl.BlockSpec((9, 2 * C, C), lambda n: (0, 0, 0)),
                pl.BlockSpec((3, 2 * C, C), lambda n: (0, 0, 0)),
                pl.BlockSpec((3, C, C), lambda n: (0, 0, 0)),
                pl.BlockSpec((1, C), lambda n: (0, 0)),
                pl.BlockSpec((1, C), lambda n: (0, 0)),
            ],
            out_specs=pl.BlockSpec((1, C, P), lambda n: (n, 0, 0)),
            scratch_shapes=[
                pltpu.VMEM((RP * WP, C), bf16),
                pltpu.VMEM((RP * WP, C), bf16),
                pltpu.VMEM((YR, 2 * C), bf16),
            ]),
        compiler_params=pltpu.CompilerParams(
            dimension_semantics=("parallel",),
            vmem_limit_bytes=64 * 1024 * 1024),
    )(x_r, skip_r, wu, bvec, w1t, w2a, w2b, sh1, sh2)

    return out_r.reshape(N, C, H2, W2)


# speedup vs baseline: 2.2920x; 1.0081x over previous
import jax
import jax.numpy as jnp
from jax import lax
from jax.experimental import pallas as pl
from jax.experimental.pallas import tpu as pltpu


def kernel(x, skip, w_up, b_up, w1, b1, g1, beta1, m1, v1,
           w2, b2, g2, beta2, m2, v2):
    f32, bf16 = jnp.float32, jnp.bfloat16
    N, Cin, H, W = x.shape
    C = b_up.shape[0]
    H2, W2 = 2 * H, 2 * W
    P = H2 * W2
    GD = (H2 + 4) * W2
    M1 = (H2 + 2) * W2
    M2 = P
    eps = 1e-5

    x_r = x.reshape(N, Cin, H * W)
    skip_r = skip.reshape(N, C, P)

    wu = jnp.transpose(w_up, (0, 2, 3, 1)).reshape(Cin, 4 * C).astype(bf16)
    bvec = jnp.tile(b_up.astype(f32), 4)[None, :]

    sc1 = g1 / jnp.sqrt(v1 + eps)
    sh1 = ((b1 - m1) * sc1 + beta1).astype(f32)[None, :]
    sc2 = g2 / jnp.sqrt(v2 + eps)
    sh2 = ((b2 - m2) * sc2 + beta2).astype(f32)[None, :]

    w1t = jnp.transpose(w1, (2, 3, 1, 0)) * sc1[None, None, None, :]
    w1k = w1t.reshape(3, 3 * 2 * C, C).astype(bf16)

    w2t = jnp.transpose(w2, (2, 3, 1, 0)) * sc2[None, None, None, :]
    w2k = w2t.reshape(3, 3 * C, C).astype(bf16)

    def shift_cols(v, d):
        M = v.shape[0]
        z1 = jnp.zeros((1, v.shape[1]), v.dtype)
        si = lax.broadcasted_iota(jnp.int32, (M, 1), 0)
        cm = si - (si // W2) * W2
        if d == 1:
            sv = jnp.concatenate([v[1:], z1], axis=0)
            bad = cm == W2 - 1
        else:
            sv = jnp.concatenate([z1, v[:-1]], axis=0)
            bad = cm == 0
        return jnp.where(bad, jnp.zeros_like(sv), sv)

    def body(x_ref, skip_ref, wu_ref, bv_ref, w1_ref, w2_ref,
             sh1_ref, sh2_ref, o_ref, cu, y3):
        n = pl.program_id(0)
        first = jnp.logical_or(n == 0, n == N // 2)

        @pl.when(first)
        def _():
            cu[0:2 * W2, :] = jnp.zeros((2 * W2, 6 * C), bf16)
            cu[GD - 2 * W2:GD, :] = jnp.zeros((2 * W2, 6 * C), bf16)

        skT = jnp.transpose(skip_ref[0]).astype(bf16)
        for j in range(3):
            sv = skT if j == 1 else shift_cols(skT, j - 1)
            cu[2 * W2:2 * W2 + P, j * 2 * C:j * 2 * C + C] = sv

        xb = x_ref[0].astype(bf16)
        zc = lax.dot_general(xb, wu_ref[...],
                             (((0,), (0,)), ((), ())),
                             preferred_element_type=f32) + bv_ref[...]
        zb = zc.astype(bf16)
        for h in range(H):
            for dy in range(2):
                bl = zb[h * W:(h + 1) * W,
                        dy * 2 * C:(dy + 1) * 2 * C].reshape(2 * W, C)
                q = (2 + 2 * h + dy) * W2
                cu[q:q + W2, 3 * C:4 * C] = bl
        upv = cu[2 * W2:2 * W2 + P, 3 * C:4 * C]
        cu[2 * W2:2 * W2 + P, C:2 * C] = shift_cols(upv, -1)
        cu[2 * W2:2 * W2 + P, 5 * C:6 * C] = shift_cols(upv, 1)

        acc = None
        for kh in range(3):
            d = jnp.dot(cu[kh * W2:kh * W2 + M1, :], w1_ref[kh],
                        preferred_element_type=f32)
            acc = d if acc is None else acc + d
        y1 = jnp.maximum(acc + sh1_ref[...], 0.0)
        ri = lax.broadcasted_iota(jnp.int32, (M1, 1), 0)
        rr = ri // W2
        ok = jnp.logical_and(rr >= 1, rr < H2 + 1)
        y1b = jnp.where(ok, y1, 0.0).astype(bf16)

        for j in range(3):
            sv = y1b if j == 1 else shift_cols(y1b, j - 1)
            y3[0:M1, j * C:(j + 1) * C] = sv

        acc2 = None
        for kh in range(3):
            d = jnp.dot(y3[kh * W2:kh * W2 + M2, :], w2_ref[kh],
                        preferred_element_type=f32)
            acc2 = d if acc2 is None else acc2 + d
        y2 = jnp.maximum(acc2 + sh2_ref[...], 0.0)

        o_ref[0] = jnp.transpose(y2)

    out_r = pl.pallas_call(
        body,
        out_shape=jax.ShapeDtypeStruct((N, C, P), f32),
        grid_spec=pltpu.PrefetchScalarGridSpec(
            num_scalar_prefetch=0,
            grid=(N,),
            in_specs=[
                pl.BlockSpec((1, Cin, H * W), lambda n: (n, 0, 0)),
                pl.BlockSpec((1, C, P), lambda n: (n, 0, 0)),
                pl.BlockSpec((Cin, 4 * C), lambda n: (0, 0)),
                pl.BlockSpec((1, 4 * C), lambda n: (0, 0)),
                pl.BlockSpec((3, 3 * 2 * C, C), lambda n: (0, 0, 0)),
                pl.BlockSpec((3, 3 * C, C), lambda n: (0, 0, 0)),
                pl.BlockSpec((1, C), lambda n: (0, 0)),
                pl.BlockSpec((1, C), lambda n: (0, 0)),
            ],
            out_specs=pl.BlockSpec((1, C, P), lambda n: (n, 0, 0)),
            scratch_shapes=[
                pltpu.VMEM((GD, 3 * 2 * C), bf16),
                pltpu.VMEM((M1, 3 * C), bf16),
            ]),
        compiler_params=pltpu.CompilerParams(
            dimension_semantics=("parallel",),
            vmem_limit_bytes=64 * 1024 * 1024),
    )(x_r, skip_r, wu, bvec, w1k, w2k, sh1, sh2)

    return out_r.reshape(N, C, H2, W2)


# speedup vs baseline: 2.4440x; 1.0663x over previous
import jax
import jax.numpy as jnp
from jax import lax
from jax.experimental import pallas as pl
from jax.experimental.pallas import tpu as pltpu


def kernel(x, skip, w_up, b_up, w1, b1, g1, beta1, m1, v1,
           w2, b2, g2, beta2, m2, v2):
    f32, bf16 = jnp.float32, jnp.bfloat16
    N, Cin, H, W = x.shape
    C = b_up.shape[0]
    H2, W2 = 2 * H, 2 * W
    P = H2 * W2
    GD = (H2 + 4) * W2
    M1 = (H2 + 2) * W2
    M2 = P
    eps = 1e-5

    x_r = x.reshape(N, Cin, H * W)
    skip_r = skip.reshape(N, C, P)

    wu = jnp.transpose(w_up, (0, 2, 3, 1)).reshape(Cin, 4 * C).astype(bf16)
    bvec = jnp.tile(b_up.astype(f32), 4)[None, :]

    sc1 = g1 / jnp.sqrt(v1 + eps)
    sh1 = ((b1 - m1) * sc1 + beta1).astype(f32)[None, :]
    sc2 = g2 / jnp.sqrt(v2 + eps)
    sh2 = ((b2 - m2) * sc2 + beta2).astype(f32)[None, :]

    w1t = jnp.transpose(w1, (2, 3, 1, 0)) * sc1[None, None, None, :]
    w1k = w1t.reshape(3, 3 * 2 * C, C).astype(bf16)

    w2t = jnp.transpose(w2, (2, 3, 1, 0)) * sc2[None, None, None, :]
    w2k = w2t.reshape(3, 3 * C, C).astype(bf16)

    def shift_cols(v, d):
        M = v.shape[0]
        z1 = jnp.zeros((1, v.shape[1]), v.dtype)
        si = lax.broadcasted_iota(jnp.int32, (M, 1), 0)
        cm = si - (si // W2) * W2
        if d == 1:
            sv = jnp.concatenate([v[1:], z1], axis=0)
            bad = cm == W2 - 1
        else:
            sv = jnp.concatenate([z1, v[:-1]], axis=0)
            bad = cm == 0
        return jnp.where(bad, jnp.zeros_like(sv), sv)

    NB = 2
    NS = N // NB

    def body(x_ref, skip_ref, wu_ref, bv_ref, w1_ref, w2_ref,
             sh1_ref, sh2_ref, o_ref, cu, y3):
        n = pl.program_id(0)
        first = jnp.logical_or(n == 0, n == NS // 2)

        @pl.when(first)
        def _():
            cu[0:2 * W2, :] = jnp.zeros((2 * W2, 6 * C), bf16)
            cu[GD - 2 * W2:GD, :] = jnp.zeros((2 * W2, 6 * C), bf16)

        for img in range(NB):
          skT = jnp.transpose(skip_ref[img]).astype(bf16)
          for j in range(3):
              sv = skT if j == 1 else shift_cols(skT, j - 1)
              cu[2 * W2:2 * W2 + P, j * 2 * C:j * 2 * C + C] = sv

          xb = x_ref[img].astype(bf16)
          zc = lax.dot_general(xb, wu_ref[...],
                               (((0,), (0,)), ((), ())),
                               preferred_element_type=f32) + bv_ref[...]
          zb = zc.astype(bf16)
          for h in range(H):
              for dy in range(2):
                  bl = zb[h * W:(h + 1) * W,
                          dy * 2 * C:(dy + 1) * 2 * C].reshape(2 * W, C)
                  q = (2 + 2 * h + dy) * W2
                  cu[q:q + W2, 3 * C:4 * C] = bl
          upv = cu[2 * W2:2 * W2 + P, 3 * C:4 * C]
          cu[2 * W2:2 * W2 + P, C:2 * C] = shift_cols(upv, -1)
          cu[2 * W2:2 * W2 + P, 5 * C:6 * C] = shift_cols(upv, 1)

          acc = None
          for kh in range(3):
              d = jnp.dot(cu[kh * W2:kh * W2 + M1, :], w1_ref[kh],
                          preferred_element_type=f32)
              acc = d if acc is None else acc + d
          y1 = jnp.maximum(acc + sh1_ref[...], 0.0)
          ri = lax.broadcasted_iota(jnp.int32, (M1, 1), 0)
          rr = ri // W2
          ok = jnp.logical_and(rr >= 1, rr < H2 + 1)
          y1b = jnp.where(ok, y1, 0.0).astype(bf16)

          for j in range(3):
              sv = y1b if j == 1 else shift_cols(y1b, j - 1)
              y3[0:M1, j * C:(j + 1) * C] = sv

          acc2 = None
          for kh in range(3):
              d = jnp.dot(y3[kh * W2:kh * W2 + M2, :], w2_ref[kh],
                          preferred_element_type=f32)
              acc2 = d if acc2 is None else acc2 + d
          y2 = jnp.maximum(acc2 + sh2_ref[...], 0.0)

          o_ref[img] = jnp.transpose(y2)

    out_r = pl.pallas_call(
        body,
        out_shape=jax.ShapeDtypeStruct((N, C, P), f32),
        grid_spec=pltpu.PrefetchScalarGridSpec(
            num_scalar_prefetch=0,
            grid=(N // 2,),
            in_specs=[
                pl.BlockSpec((2, Cin, H * W), lambda n: (n, 0, 0)),
                pl.BlockSpec((2, C, P), lambda n: (n, 0, 0)),
                pl.BlockSpec((Cin, 4 * C), lambda n: (0, 0)),
                pl.BlockSpec((1, 4 * C), lambda n: (0, 0)),
                pl.BlockSpec((3, 3 * 2 * C, C), lambda n: (0, 0, 0)),
                pl.BlockSpec((3, 3 * C, C), lambda n: (0, 0, 0)),
                pl.BlockSpec((1, C), lambda n: (0, 0)),
                pl.BlockSpec((1, C), lambda n: (0, 0)),
            ],
            out_specs=pl.BlockSpec((2, C, P), lambda n: (n, 0, 0)),
            scratch_shapes=[
                pltpu.VMEM((GD, 3 * 2 * C), bf16),
                pltpu.VMEM((M1, 3 * C), bf16),
            ]),
        compiler_params=pltpu.CompilerParams(
            dimension_semantics=("parallel",),
            vmem_limit_bytes=64 * 1024 * 1024),
    )(x_r, skip_r, wu, bvec, w1k, w2k, sh1, sh2)

    return out_r.reshape(N, C, H2, W2)
